# baseline (speedup 1.0000x reference)
import functools

import jax
import jax.numpy as jnp
from jax import lax
from jax.experimental import pallas as pl
from jax.experimental.pallas import tpu as pltpu
from jax.experimental.pallas import tpu_sc as plsc

N = 10000
D = 128
NC = 2
NS = 16
L = 16
C = 128
E_RAW = 320000
G = 8
NG0 = 20
NG1 = 0
DEPTH = 2
E_PAD = NS * G * C * (NG0 + NG1)
NSTRIPE = -(-N // C)
N_PAD = NSTRIPE * C
DEG_W = D

@functools.lru_cache(maxsize=None)
def _get_mesh():
    return plsc.VectorSubcoreMesh(core_axis_name="c", subcore_axis_name="s",
                                  num_cores=NC, num_subcores=NS)


def _fill(ref, rows, cols, val):
    @pl.loop(0, rows)
    def _(r):
        @pl.loop(0, cols, step=L)
        def _(c0):
            ref[r, pl.ds(c0, L)] = jnp.full((L,), val, jnp.float32)


def _deg_kernel(dst2):
    @functools.partial(
        pl.kernel,
        out_type=jax.ShapeDtypeStruct((NC, N, DEG_W), jnp.float32),
        mesh=_get_mesh(),
        scratch_types=[
            pltpu.VMEM((G, C), jnp.int32),
            pltpu.VMEM((C, DEG_W), jnp.float32),
            pltpu.VMEM_SHARED((N_PAD, DEG_W), jnp.float32),
        ],
    )
    def k(dst_hbm, out_hbm, dstg, ones_v, dacc):
        cid = lax.axis_index("c")
        sid = lax.axis_index("s")
        ngc = jnp.where(cid == 0, NG0, NG1)
        grp0 = jnp.where(cid == 0, sid * NG0, NS * NG0 + sid * NG1)
        _fill(ones_v, C, DEG_W, 0.0)
        nk = -(-NSTRIPE // NS)
        for kk in range(nk):
            s = jnp.minimum(sid + NS * kk, NSTRIPE - 1)
            pltpu.sync_copy(ones_v, dacc.at[pl.ds(s * C, C)])
        _fill(ones_v, C, DEG_W, 1.0)
        plsc.subcore_barrier()

        @pl.loop(0, ngc)
        def _(g):
            crow = (grp0 + g) * G
            pltpu.sync_copy(dst_hbm.at[pl.ds(crow, G)], dstg)
            for j in range(G):
                pltpu.sync_copy(ones_v, dacc.at[dstg.at[j]], add=True)

        plsc.subcore_barrier()
        for kk in range(nk):
            s = jnp.minimum(sid + NS * kk, NSTRIPE - 1)
            start = jnp.minimum(s * C, N - C)
            pltpu.sync_copy(dacc.at[pl.ds(start, C)],
                            out_hbm.at[cid, pl.ds(start, C)])

    return k(dst2)


def _agg_kernel(y, src2, dst2):
    @functools.partial(
        pl.kernel,
        out_type=jax.ShapeDtypeStruct((NC, N, D), jnp.float32),
        mesh=_get_mesh(),
        scratch_types=[
            pltpu.VMEM((G, C), jnp.int32),
            pltpu.VMEM((G, C), jnp.int32),
        ] + [pltpu.VMEM((C, D), jnp.float32)] * DEPTH + [
            pltpu.VMEM_SHARED((N_PAD, D), jnp.float32),
        ] + [pltpu.SemaphoreType.DMA] * DEPTH,
    )
    def k(y_hbm, src_hbm, dst_hbm, out_hbm, srcg, dstg, *rest):
        bufs = rest[:DEPTH]
        zacc = rest[DEPTH]
        sems = rest[DEPTH + 1:]
        cid = lax.axis_index("c")
        sid = lax.axis_index("s")
        ngc = jnp.where(cid == 0, NG0, NG1)
        grp0 = jnp.where(cid == 0, sid * NG0, NS * NG0 + sid * NG1)
        _fill(bufs[0], C, D, 0.0)
        nk = -(-NSTRIPE // NS)
        for kk in range(nk):
            s = jnp.minimum(sid + NS * kk, NSTRIPE - 1)
            pltpu.sync_copy(bufs[0], zacc.at[pl.ds(s * C, C)])
        plsc.subcore_barrier()

        @pl.loop(0, ngc)
        def _(g):
            crow = (grp0 + g) * G
            pltpu.sync_copy(src_hbm.at[pl.ds(crow, G)], srcg)
            pltpu.sync_copy(dst_hbm.at[pl.ds(crow, G)], dstg)
            hs = [pltpu.async_copy(y_hbm.at[srcg.at[j]], bufs[j % DEPTH],
                                   sems[j % DEPTH])
                  for j in range(DEPTH - 1)]
            for j in range(G):
                jn = j + DEPTH - 1
                if jn < G:
                    hs.append(pltpu.async_copy(y_hbm.at[srcg.at[jn]],
                                               bufs[jn % DEPTH],
                                               sems[jn % DEPTH]))
                hs[j].wait()
                pltpu.sync_copy(bufs[j % DEPTH], zacc.at[dstg.at[j]], add=True)

        plsc.subcore_barrier()
        for kk in range(nk):
            s = jnp.minimum(sid + NS * kk, NSTRIPE - 1)
            start = jnp.minimum(s * C, N - C)
            pltpu.sync_copy(zacc.at[pl.ds(start, C)],
                            out_hbm.at[cid, pl.ds(start, C)])

    return k(y, src2, dst2)


R = 1000


def _prep_body(dp_ref, x0_ref, y1_ref, dinv_ref):
    deg = dp_ref[0, :, 0:1] + dp_ref[1, :, 0:1] + 1.0
    dinv = lax.rsqrt(deg)
    y1_ref[...] = x0_ref[...] * dinv
    dinv_ref[...] = jnp.broadcast_to(dinv, dinv_ref.shape)


def _mid_body(p_ref, y1_ref, dinv8_ref, w1_ref, b1_ref, y2_ref):
    dinv = dinv8_ref[:, 0:1]
    ax = (p_ref[0] + p_ref[1] + y1_ref[...]) * dinv
    h = jnp.dot(ax, w1_ref[...], preferred_element_type=jnp.float32)
    x = jnp.maximum(h + b1_ref[...], 0.0)
    y2_ref[...] = x * dinv


def _softmax(v):
    m = jnp.max(v, axis=-1, keepdims=True)
    e = jnp.exp(v - m)
    return e / jnp.sum(e, axis=-1, keepdims=True)


def _head_body(q_ref, y2_ref, dinv8_ref, x0_ref,
               wt_ref, bt_ref, lt_w_ref, lt_b_ref,
               ws_ref, bs_ref, ls_w_ref, ls_b_ref,
               wa_ref, ba_ref, la_w_ref, la_b_ref,
               ltf_w_ref, ltf_b_ref,
               ot_ref, os_ref, otm_ref, oa_ref):
    dinv = dinv8_ref[:, 0:1]
    agg = (q_ref[0] + q_ref[1] + y2_ref[...]) * dinv

    def head(w_ref, b_ref, l_w_ref, l_b_ref):
        c = jnp.dot(agg, w_ref[...], preferred_element_type=jnp.float32)
        c = c + b_ref[...]
        t = jnp.dot(c, l_w_ref[...], preferred_element_type=jnp.float32)
        return _softmax(t + l_b_ref[...])

    ot_ref[...] = head(wt_ref, bt_ref, lt_w_ref, lt_b_ref)
    os_ref[...] = head(ws_ref, bs_ref, ls_w_ref, ls_b_ref)
    oa_ref[...] = head(wa_ref, ba_ref, la_w_ref, la_b_ref)
    tm = jnp.dot(x0_ref[...], ltf_w_ref[...], preferred_element_type=jnp.float32)
    otm_ref[...] = _softmax(tm + ltf_b_ref[...])


def _row_spec(shape):
    nd = len(shape)
    if nd == 3:
        return pl.BlockSpec((shape[0], R, shape[2]), lambda i: (0, i, 0))
    return pl.BlockSpec((R, shape[1]), lambda i: (i, 0))


def _full_spec(shape):
    return pl.BlockSpec(shape, lambda i: tuple(0 for _ in shape))


def _tc_call(body, ins, out_shapes):
    specs = [_row_spec(a.shape) if a.shape[-2] == N else _full_spec(a.shape)
             for a in ins]
    out_specs = [_row_spec(s.shape) for s in out_shapes]
    return pl.pallas_call(
        body,
        grid=(N // R,),
        in_specs=specs,
        out_specs=out_specs if len(out_specs) > 1 else out_specs[0],
        out_shape=out_shapes if len(out_shapes) > 1 else out_shapes[0],
    )(*ins)


def kernel(x0, edge_index, W1, b1, Wt, bt, Ws, bs, Wtm, btm, Wa, ba,
           Lt_W, Lt_b, Ls_W, Ls_b, Ltf_W, Ltf_b, La_W, La_b):
    src = edge_index[0].astype(jnp.int32)
    dst = edge_index[1].astype(jnp.int32)
    pad = E_PAD - E_RAW
    src_p = jnp.concatenate([src, jnp.zeros((pad,), jnp.int32)]).reshape(-1, C)
    dst_p = jnp.concatenate([dst, jnp.full((pad,), N, jnp.int32)]).reshape(-1, C)

    dp = _deg_kernel(dst_p)
    y1, dinv8 = _tc_call(_prep_body, [dp, x0],
                         [jax.ShapeDtypeStruct((N, D), jnp.float32),
                          jax.ShapeDtypeStruct((N, 8), jnp.float32)])
    p = _agg_kernel(y1, src_p, dst_p)
    y2 = _tc_call(_mid_body, [p, y1, dinv8, W1, b1.reshape(1, -1)],
                  [jax.ShapeDtypeStruct((N, D), jnp.float32)])
    q = _agg_kernel(y2, src_p, dst_p)
    heads = _tc_call(
        _head_body,
        [q, y2, dinv8, x0,
         Wt, bt.reshape(1, -1), Lt_W, Lt_b.reshape(1, -1),
         Ws, bs.reshape(1, -1), Ls_W, Ls_b.reshape(1, -1),
         Wa, ba.reshape(1, -1), La_W, La_b.reshape(1, -1),
         Ltf_W, Ltf_b.reshape(1, -1)],
        [jax.ShapeDtypeStruct((N, 16), jnp.float32),
         jax.ShapeDtypeStruct((N, 32), jnp.float32),
         jax.ShapeDtypeStruct((N, 8), jnp.float32),
         jax.ShapeDtypeStruct((N, 64), jnp.float32)])
    out_type, out_school, out_time, out_author = heads
    return (out_type, out_school, out_time, out_author)


# speedup vs baseline: 3.3363x; 3.3363x over previous
import functools

import jax
import jax.numpy as jnp
from jax import lax
from jax.experimental import pallas as pl
from jax.experimental.pallas import tpu as pltpu
from jax.experimental.pallas import tpu_sc as plsc

N = 10000
D = 128
NC = 2
NS = 16
L = 16
C = 128
E_RAW = 320000
G = 8
NG0 = 10
NG1 = 10
DEPTH = 2
E_PAD = NS * G * C * (NG0 + NG1)
NSTRIPE = -(-N // C)
N_PAD = NSTRIPE * C
DEG_W = D

@functools.lru_cache(maxsize=None)
def _get_mesh():
    return plsc.VectorSubcoreMesh(core_axis_name="c", subcore_axis_name="s",
                                  num_cores=NC, num_subcores=NS)


def _fill(ref, rows, cols, val):
    @pl.loop(0, rows)
    def _(r):
        @pl.loop(0, cols, step=L)
        def _(c0):
            ref[r, pl.ds(c0, L)] = jnp.full((L,), val, jnp.float32)


def _deg_kernel(dst2):
    @functools.partial(
        pl.kernel,
        out_type=jax.ShapeDtypeStruct((NC, N, DEG_W), jnp.float32),
        mesh=_get_mesh(),
        scratch_types=[
            pltpu.VMEM((G, C), jnp.int32),
            pltpu.VMEM((C, DEG_W), jnp.float32),
            pltpu.VMEM_SHARED((N_PAD, DEG_W), jnp.float32),
        ],
    )
    def k(dst_hbm, out_hbm, dstg, ones_v, dacc):
        cid = lax.axis_index("c")
        sid = lax.axis_index("s")
        ngc = jnp.where(cid == 0, NG0, NG1)
        grp0 = jnp.where(cid == 0, sid * NG0, NS * NG0 + sid * NG1)
        _fill(ones_v, C, DEG_W, 0.0)
        nk = -(-NSTRIPE // NS)
        for kk in range(nk):
            s = jnp.minimum(sid + NS * kk, NSTRIPE - 1)
            pltpu.sync_copy(ones_v, dacc.at[pl.ds(s * C, C)])
        _fill(ones_v, C, DEG_W, 1.0)
        plsc.subcore_barrier()

        @pl.loop(0, ngc)
        def _(g):
            crow = (grp0 + g) * G
            pltpu.sync_copy(dst_hbm.at[pl.ds(crow, G)], dstg)
            for j in range(G):
                pltpu.sync_copy(ones_v, dacc.at[dstg.at[j]], add=True)

        plsc.subcore_barrier()
        for kk in range(nk):
            s = jnp.minimum(sid + NS * kk, NSTRIPE - 1)
            start = jnp.minimum(s * C, N - C)
            pltpu.sync_copy(dacc.at[pl.ds(start, C)],
                            out_hbm.at[cid, pl.ds(start, C)])

    return k(dst2)


def _agg_kernel(y, src2, dst2):
    @functools.partial(
        pl.kernel,
        out_type=jax.ShapeDtypeStruct((NC, N, D), jnp.float32),
        mesh=_get_mesh(),
        scratch_types=[
            pltpu.VMEM((G, C), jnp.int32),
            pltpu.VMEM((G, C), jnp.int32),
        ] + [pltpu.VMEM((C, D), jnp.float32)] * DEPTH + [
            pltpu.VMEM_SHARED((N_PAD, D), jnp.float32),
        ] + [pltpu.SemaphoreType.DMA] * DEPTH,
    )
    def k(y_hbm, src_hbm, dst_hbm, out_hbm, srcg, dstg, *rest):
        bufs = rest[:DEPTH]
        zacc = rest[DEPTH]
        sems = rest[DEPTH + 1:]
        cid = lax.axis_index("c")
        sid = lax.axis_index("s")
        ngc = jnp.where(cid == 0, NG0, NG1)
        grp0 = jnp.where(cid == 0, sid * NG0, NS * NG0 + sid * NG1)
        _fill(bufs[0], C, D, 0.0)
        nk = -(-NSTRIPE // NS)
        for kk in range(nk):
            s = jnp.minimum(sid + NS * kk, NSTRIPE - 1)
            pltpu.sync_copy(bufs[0], zacc.at[pl.ds(s * C, C)])
        plsc.subcore_barrier()

        @pl.loop(0, ngc)
        def _(g):
            crow = (grp0 + g) * G
            pltpu.sync_copy(src_hbm.at[pl.ds(crow, G)], srcg)
            pltpu.sync_copy(dst_hbm.at[pl.ds(crow, G)], dstg)
            hs = [pltpu.async_copy(y_hbm.at[srcg.at[j]], bufs[j % DEPTH],
                                   sems[j % DEPTH])
                  for j in range(DEPTH - 1)]
            for j in range(G):
                jn = j + DEPTH - 1
                if jn < G:
                    hs.append(pltpu.async_copy(y_hbm.at[srcg.at[jn]],
                                               bufs[jn % DEPTH],
                                               sems[jn % DEPTH]))
                hs[j].wait()
                pltpu.sync_copy(bufs[j % DEPTH], zacc.at[dstg.at[j]], add=True)

        plsc.subcore_barrier()
        for kk in range(nk):
            s = jnp.minimum(sid + NS * kk, NSTRIPE - 1)
            start = jnp.minimum(s * C, N - C)
            pltpu.sync_copy(zacc.at[pl.ds(start, C)],
                            out_hbm.at[cid, pl.ds(start, C)])

    return k(y, src2, dst2)


R = 1000


def _prep_body(dp_ref, x0_ref, y1_ref, dinv_ref):
    deg = dp_ref[0, :, 0:1] + dp_ref[1, :, 0:1] + 1.0
    dinv = lax.rsqrt(deg)
    y1_ref[...] = x0_ref[...] * dinv
    dinv_ref[...] = jnp.broadcast_to(dinv, dinv_ref.shape)


def _mid_body(p_ref, y1_ref, dinv8_ref, w1_ref, b1_ref, y2_ref):
    dinv = dinv8_ref[:, 0:1]
    ax = (p_ref[0] + p_ref[1] + y1_ref[...]) * dinv
    h = jnp.dot(ax, w1_ref[...], preferred_element_type=jnp.float32)
    x = jnp.maximum(h + b1_ref[...], 0.0)
    y2_ref[...] = x * dinv


def _softmax(v):
    m = jnp.max(v, axis=-1, keepdims=True)
    e = jnp.exp(v - m)
    return e / jnp.sum(e, axis=-1, keepdims=True)


def _head_body(q_ref, y2_ref, dinv8_ref, x0_ref,
               wt_ref, bt_ref, lt_w_ref, lt_b_ref,
               ws_ref, bs_ref, ls_w_ref, ls_b_ref,
               wa_ref, ba_ref, la_w_ref, la_b_ref,
               ltf_w_ref, ltf_b_ref,
               ot_ref, os_ref, otm_ref, oa_ref):
    dinv = dinv8_ref[:, 0:1]
    agg = (q_ref[0] + q_ref[1] + y2_ref[...]) * dinv

    def head(w_ref, b_ref, l_w_ref, l_b_ref):
        c = jnp.dot(agg, w_ref[...], preferred_element_type=jnp.float32)
        c = c + b_ref[...]
        t = jnp.dot(c, l_w_ref[...], preferred_element_type=jnp.float32)
        return _softmax(t + l_b_ref[...])

    ot_ref[...] = head(wt_ref, bt_ref, lt_w_ref, lt_b_ref)
    os_ref[...] = head(ws_ref, bs_ref, ls_w_ref, ls_b_ref)
    oa_ref[...] = head(wa_ref, ba_ref, la_w_ref, la_b_ref)
    tm = jnp.dot(x0_ref[...], ltf_w_ref[...], preferred_element_type=jnp.float32)
    otm_ref[...] = _softmax(tm + ltf_b_ref[...])


def _row_spec(shape):
    nd = len(shape)
    if nd == 3:
        return pl.BlockSpec((shape[0], R, shape[2]), lambda i: (0, i, 0))
    return pl.BlockSpec((R, shape[1]), lambda i: (i, 0))


def _full_spec(shape):
    return pl.BlockSpec(shape, lambda i: tuple(0 for _ in shape))


def _tc_call(body, ins, out_shapes):
    specs = [_row_spec(a.shape) if a.shape[-2] == N else _full_spec(a.shape)
             for a in ins]
    out_specs = [_row_spec(s.shape) for s in out_shapes]
    return pl.pallas_call(
        body,
        grid=(N // R,),
        in_specs=specs,
        out_specs=out_specs if len(out_specs) > 1 else out_specs[0],
        out_shape=out_shapes if len(out_shapes) > 1 else out_shapes[0],
    )(*ins)


def kernel(x0, edge_index, W1, b1, Wt, bt, Ws, bs, Wtm, btm, Wa, ba,
           Lt_W, Lt_b, Ls_W, Ls_b, Ltf_W, Ltf_b, La_W, La_b):
    src = edge_index[0].astype(jnp.int32)
    dst = edge_index[1].astype(jnp.int32)
    pad = E_PAD - E_RAW
    pad_i = jnp.arange(pad, dtype=jnp.int32)
    src_p = jnp.concatenate([src, pad_i % N]).reshape(-1, C)
    dst_p = jnp.concatenate([dst, N + pad_i % (N_PAD - N)]).reshape(-1, C)

    dp = _deg_kernel(dst_p)
    y1, dinv8 = _tc_call(_prep_body, [dp, x0],
                         [jax.ShapeDtypeStruct((N, D), jnp.float32),
                          jax.ShapeDtypeStruct((N, 8), jnp.float32)])
    p = _agg_kernel(y1, src_p, dst_p)
    y2 = _tc_call(_mid_body, [p, y1, dinv8, W1, b1.reshape(1, -1)],
                  [jax.ShapeDtypeStruct((N, D), jnp.float32)])
    q = _agg_kernel(y2, src_p, dst_p)
    heads = _tc_call(
        _head_body,
        [q, y2, dinv8, x0,
         Wt, bt.reshape(1, -1), Lt_W, Lt_b.reshape(1, -1),
         Ws, bs.reshape(1, -1), Ls_W, Ls_b.reshape(1, -1),
         Wa, ba.reshape(1, -1), La_W, La_b.reshape(1, -1),
         Ltf_W, Ltf_b.reshape(1, -1)],
        [jax.ShapeDtypeStruct((N, 16), jnp.float32),
         jax.ShapeDtypeStruct((N, 32), jnp.float32),
         jax.ShapeDtypeStruct((N, 8), jnp.float32),
         jax.ShapeDtypeStruct((N, 64), jnp.float32)])
    out_type, out_school, out_time, out_author = heads
    return (out_type, out_school, out_time, out_author)


# speedup vs baseline: 3.4772x; 1.0422x over previous
import dataclasses
import functools

import jax
import jax.numpy as jnp
import numpy as np
from jax import lax
from jax.experimental import pallas as pl
from jax.experimental.pallas import tpu as pltpu
from jax.experimental.pallas import tpu_sc as plsc

N = 10000
D = 128
NC = 2
NS = 16
L = 16
C = 128
E_RAW = 320000
G = 8
NG0 = 10
NG1 = 10
DEPTH = 2
E_PAD = NS * G * C * (NG0 + NG1)
NSTRIPE = -(-N // C)
N_PAD = NSTRIPE * C
DEG_W = D

@functools.lru_cache(maxsize=None)
def _get_mesh():
    return plsc.VectorSubcoreMesh(core_axis_name="c", subcore_axis_name="s",
                                  num_cores=NC, num_subcores=NS)


def _fill(ref, rows, cols, val):
    @pl.loop(0, rows)
    def _(r):
        @pl.loop(0, cols, step=L)
        def _(c0):
            ref[r, pl.ds(c0, L)] = jnp.full((L,), val, jnp.float32)


def _deg_kernel(dst2):
    @functools.partial(
        pl.kernel,
        out_type=jax.ShapeDtypeStruct((NC * NS, N_PAD), jnp.float32),
        mesh=_get_mesh(),
        scratch_types=[
            pltpu.VMEM((G, C), jnp.int32),
            pltpu.VMEM((N_PAD,), jnp.float32),
        ],
        compiler_params=dataclasses.replace(pltpu.CompilerParams(),
                                            needs_layout_passes=False),
    )
    def k(dst_hbm, out_hbm, dstg, acc):
        cid = lax.axis_index("c")
        sid = lax.axis_index("s")
        wid = cid * NS + sid
        ngc = jnp.where(cid == 0, NG0, NG1)
        grp0 = jnp.where(cid == 0, sid * NG0, NS * NG0 + sid * NG1)

        @pl.loop(0, N_PAD, step=L)
        def _(i):
            acc[pl.ds(i, L)] = jnp.zeros((L,), jnp.float32)

        ones = jnp.ones((L,), jnp.float32)

        @pl.loop(0, ngc)
        def _(g):
            crow = (grp0 + g) * G
            pltpu.sync_copy(dst_hbm.at[pl.ds(crow, G)], dstg)
            for j in range(G):
                for c0 in range(0, C, L):
                    idx16 = dstg[j, pl.ds(c0, L)]
                    plsc.addupdate_scatter(acc, [idx16], ones)

        pltpu.sync_copy(acc, out_hbm.at[wid])

    return k(dst2)


def _agg_kernel(y, src2, dst2):
    @functools.partial(
        pl.kernel,
        out_type=jax.ShapeDtypeStruct((NC, N, D), jnp.float32),
        mesh=_get_mesh(),
        scratch_types=[
            pltpu.VMEM((G, C), jnp.int32),
            pltpu.VMEM((G, C), jnp.int32),
        ] + [pltpu.VMEM((C, D), jnp.float32)] * DEPTH + [
            pltpu.VMEM_SHARED((N_PAD, D), jnp.float32),
        ] + [pltpu.SemaphoreType.DMA] * DEPTH,
    )
    def k(y_hbm, src_hbm, dst_hbm, out_hbm, srcg, dstg, *rest):
        bufs = rest[:DEPTH]
        zacc = rest[DEPTH]
        sems = rest[DEPTH + 1:]
        cid = lax.axis_index("c")
        sid = lax.axis_index("s")
        ngc = jnp.where(cid == 0, NG0, NG1)
        grp0 = jnp.where(cid == 0, sid * NG0, NS * NG0 + sid * NG1)
        _fill(bufs[0], C, D, 0.0)
        nk = -(-NSTRIPE // NS)
        for kk in range(nk):
            s = jnp.minimum(sid + NS * kk, NSTRIPE - 1)
            pltpu.sync_copy(bufs[0], zacc.at[pl.ds(s * C, C)])
        plsc.subcore_barrier()

        @pl.loop(0, ngc)
        def _(g):
            crow = (grp0 + g) * G
            pltpu.sync_copy(src_hbm.at[pl.ds(crow, G)], srcg)
            pltpu.sync_copy(dst_hbm.at[pl.ds(crow, G)], dstg)
            hs = [pltpu.async_copy(y_hbm.at[srcg.at[j]], bufs[j % DEPTH],
                                   sems[j % DEPTH])
                  for j in range(DEPTH - 1)]
            for j in range(G):
                jn = j + DEPTH - 1
                if jn < G:
                    hs.append(pltpu.async_copy(y_hbm.at[srcg.at[jn]],
                                               bufs[jn % DEPTH],
                                               sems[jn % DEPTH]))
                hs[j].wait()
                pltpu.sync_copy(bufs[j % DEPTH], zacc.at[dstg.at[j]], add=True)

        plsc.subcore_barrier()
        for kk in range(nk):
            s = jnp.minimum(sid + NS * kk, NSTRIPE - 1)
            start = jnp.minimum(s * C, N - C)
            pltpu.sync_copy(zacc.at[pl.ds(start, C)],
                            out_hbm.at[cid, pl.ds(start, C)])

    return k(y, src2, dst2)


R = 1000


def _dinv_body(dp_ref, dinv_ref):
    deg = jnp.sum(dp_ref[...], axis=0, keepdims=True) + 1.0
    dinv = lax.rsqrt(deg).T
    dinv_ref[...] = jnp.broadcast_to(dinv, dinv_ref.shape)


def _dinv_call(dp):
    return pl.pallas_call(
        _dinv_body,
        grid=(N_PAD // C,),
        in_specs=[pl.BlockSpec((NC * NS, C), lambda i: (0, i))],
        out_specs=pl.BlockSpec((C, 8), lambda i: (i, 0)),
        out_shape=jax.ShapeDtypeStruct((N_PAD, 8), jnp.float32),
    )(dp)


def _prep_body(dinv8_ref, x0_ref, y1_ref):
    y1_ref[...] = x0_ref[...] * dinv8_ref[:, 0:1]


def _mid_body(p_ref, y1_ref, dinv8_ref, w1_ref, b1_ref, y2_ref):
    dinv = dinv8_ref[:, 0:1]
    ax = (p_ref[0] + p_ref[1] + y1_ref[...]) * dinv
    h = jnp.dot(ax, w1_ref[...], preferred_element_type=jnp.float32)
    x = jnp.maximum(h + b1_ref[...], 0.0)
    y2_ref[...] = x * dinv


def _softmax(v):
    m = jnp.max(v, axis=-1, keepdims=True)
    e = jnp.exp(v - m)
    return e / jnp.sum(e, axis=-1, keepdims=True)


def _head_body(q_ref, y2_ref, dinv8_ref, x0_ref,
               wt_ref, bt_ref, lt_w_ref, lt_b_ref,
               ws_ref, bs_ref, ls_w_ref, ls_b_ref,
               wa_ref, ba_ref, la_w_ref, la_b_ref,
               ltf_w_ref, ltf_b_ref,
               ot_ref, os_ref, otm_ref, oa_ref):
    dinv = dinv8_ref[:, 0:1]
    agg = (q_ref[0] + q_ref[1] + y2_ref[...]) * dinv

    def head(w_ref, b_ref, l_w_ref, l_b_ref):
        c = jnp.dot(agg, w_ref[...], preferred_element_type=jnp.float32)
        c = c + b_ref[...]
        t = jnp.dot(c, l_w_ref[...], preferred_element_type=jnp.float32)
        return _softmax(t + l_b_ref[...])

    ot_ref[...] = head(wt_ref, bt_ref, lt_w_ref, lt_b_ref)
    os_ref[...] = head(ws_ref, bs_ref, ls_w_ref, ls_b_ref)
    oa_ref[...] = head(wa_ref, ba_ref, la_w_ref, la_b_ref)
    tm = jnp.dot(x0_ref[...], ltf_w_ref[...], preferred_element_type=jnp.float32)
    otm_ref[...] = _softmax(tm + ltf_b_ref[...])


def _row_spec(shape):
    nd = len(shape)
    if nd == 3:
        return pl.BlockSpec((shape[0], R, shape[2]), lambda i: (0, i, 0))
    return pl.BlockSpec((R, shape[1]), lambda i: (i, 0))


def _full_spec(shape):
    return pl.BlockSpec(shape, lambda i: tuple(0 for _ in shape))


def _tc_call(body, ins, out_shapes):
    specs = [_row_spec(a.shape) if a.shape[-2] in (N, N_PAD)
             else _full_spec(a.shape) for a in ins]
    out_specs = [_row_spec(s.shape) for s in out_shapes]
    return pl.pallas_call(
        body,
        grid=(N // R,),
        in_specs=specs,
        out_specs=out_specs if len(out_specs) > 1 else out_specs[0],
        out_shape=out_shapes if len(out_shapes) > 1 else out_shapes[0],
    )(*ins)


def kernel(x0, edge_index, W1, b1, Wt, bt, Ws, bs, Wtm, btm, Wa, ba,
           Lt_W, Lt_b, Ls_W, Ls_b, Ltf_W, Ltf_b, La_W, La_b):
    src = edge_index[0].astype(jnp.int32)
    dst = edge_index[1].astype(jnp.int32)
    pad = E_PAD - E_RAW
    pad_i = np.arange(pad, dtype=np.int32)
    src_p = jnp.concatenate([src, jnp.asarray(pad_i % N)]).reshape(-1, C)
    dst_p = jnp.concatenate(
        [dst, jnp.asarray(N + pad_i % (N_PAD - N))]).reshape(-1, C)

    dp = _deg_kernel(dst_p)
    dinv8 = _dinv_call(dp)
    y1 = _tc_call(_prep_body, [dinv8, x0],
                  [jax.ShapeDtypeStruct((N, D), jnp.float32)])
    p = _agg_kernel(y1, src_p, dst_p)
    y2 = _tc_call(_mid_body, [p, y1, dinv8, W1, b1.reshape(1, -1)],
                  [jax.ShapeDtypeStruct((N, D), jnp.float32)])
    q = _agg_kernel(y2, src_p, dst_p)
    heads = _tc_call(
        _head_body,
        [q, y2, dinv8, x0,
         Wt, bt.reshape(1, -1), Lt_W, Lt_b.reshape(1, -1),
         Ws, bs.reshape(1, -1), Ls_W, Ls_b.reshape(1, -1),
         Wa, ba.reshape(1, -1), La_W, La_b.reshape(1, -1),
         Ltf_W, Ltf_b.reshape(1, -1)],
        [jax.ShapeDtypeStruct((N, 16), jnp.float32),
         jax.ShapeDtypeStruct((N, 32), jnp.float32),
         jax.ShapeDtypeStruct((N, 8), jnp.float32),
         jax.ShapeDtypeStruct((N, 64), jnp.float32)])
    out_type, out_school, out_time, out_author = heads
    return (out_type, out_school, out_time, out_author)


# speedup vs baseline: 3.9032x; 1.1225x over previous
import dataclasses
import functools

import jax
import jax.numpy as jnp
import numpy as np
from jax import lax
from jax.experimental import pallas as pl
from jax.experimental.pallas import tpu as pltpu
from jax.experimental.pallas import tpu_sc as plsc

N = 10000
D = 128
NC = 2
NS = 16
L = 16
C = 128
E_RAW = 320000
G = 8
NG0 = 10
NG1 = 10
DEPTH = 2
E_PAD = NS * G * C * (NG0 + NG1)
NSTRIPE = -(-N // C)
N_PAD = NSTRIPE * C
DEG_W = D

@functools.lru_cache(maxsize=None)
def _get_mesh():
    return plsc.VectorSubcoreMesh(core_axis_name="c", subcore_axis_name="s",
                                  num_cores=NC, num_subcores=NS)


def _fill(ref, rows, cols, val):
    @pl.loop(0, rows)
    def _(r):
        @pl.loop(0, cols, step=L)
        def _(c0):
            ref[r, pl.ds(c0, L)] = jnp.full((L,), val, jnp.float32)


def _deg_kernel(ei_p):
    @functools.partial(
        pl.kernel,
        out_type=jax.ShapeDtypeStruct((NC * NS, N_PAD), jnp.float32),
        mesh=_get_mesh(),
        scratch_types=[
            pltpu.VMEM((G, 2, C), jnp.int32),
            pltpu.VMEM((N_PAD,), jnp.float32),
        ],
        compiler_params=dataclasses.replace(pltpu.CompilerParams(),
                                            needs_layout_passes=False),
    )
    def k(ei_hbm, out_hbm, sdg, acc):
        cid = lax.axis_index("c")
        sid = lax.axis_index("s")
        wid = cid * NS + sid
        ngc = jnp.where(cid == 0, NG0, NG1)
        grp0 = jnp.where(cid == 0, sid * NG0, NS * NG0 + sid * NG1)

        @pl.loop(0, N_PAD, step=L)
        def _(i):
            acc[pl.ds(i, L)] = jnp.zeros((L,), jnp.float32)

        ones = jnp.ones((L,), jnp.float32)

        @pl.loop(0, ngc)
        def _(g):
            crow = (grp0 + g) * G
            pltpu.sync_copy(ei_hbm.at[pl.ds(crow, G)], sdg)
            for j in range(G):
                for c0 in range(0, C, L):
                    idx16 = sdg[j, 1, pl.ds(c0, L)]
                    plsc.addupdate_scatter(acc, [idx16], ones)

        pltpu.sync_copy(acc, out_hbm.at[wid])

    return k(ei_p)


def _agg_kernel(y, ei_p):
    @functools.partial(
        pl.kernel,
        out_type=jax.ShapeDtypeStruct((NC, N, D), jnp.float32),
        mesh=_get_mesh(),
        scratch_types=[
            pltpu.VMEM((G, 2, C), jnp.int32),
        ] + [pltpu.VMEM((C, D), jnp.float32)] * DEPTH + [
            pltpu.VMEM_SHARED((N_PAD, D), jnp.float32),
        ] + [pltpu.SemaphoreType.DMA] * DEPTH,
    )
    def k(y_hbm, ei_hbm, out_hbm, sdg, *rest):
        bufs = rest[:DEPTH]
        zacc = rest[DEPTH]
        sems = rest[DEPTH + 1:]
        cid = lax.axis_index("c")
        sid = lax.axis_index("s")
        ngc = jnp.where(cid == 0, NG0, NG1)
        grp0 = jnp.where(cid == 0, sid * NG0, NS * NG0 + sid * NG1)
        _fill(bufs[0], C, D, 0.0)
        nk = -(-NSTRIPE // NS)
        for kk in range(nk):
            s = jnp.minimum(sid + NS * kk, NSTRIPE - 1)
            pltpu.sync_copy(bufs[0], zacc.at[pl.ds(s * C, C)])
        plsc.subcore_barrier()

        @pl.loop(0, ngc)
        def _(g):
            crow = (grp0 + g) * G
            pltpu.sync_copy(ei_hbm.at[pl.ds(crow, G)], sdg)
            hs = [pltpu.async_copy(y_hbm.at[sdg.at[j, 0]], bufs[j % DEPTH],
                                   sems[j % DEPTH])
                  for j in range(DEPTH - 1)]
            for j in range(G):
                jn = j + DEPTH - 1
                if jn < G:
                    hs.append(pltpu.async_copy(y_hbm.at[sdg.at[jn, 0]],
                                               bufs[jn % DEPTH],
                                               sems[jn % DEPTH]))
                hs[j].wait()
                pltpu.sync_copy(bufs[j % DEPTH], zacc.at[sdg.at[j, 1]],
                                add=True)

        plsc.subcore_barrier()
        for kk in range(nk):
            s = jnp.minimum(sid + NS * kk, NSTRIPE - 1)
            start = jnp.minimum(s * C, N - C)
            pltpu.sync_copy(zacc.at[pl.ds(start, C)],
                            out_hbm.at[cid, pl.ds(start, C)])

    return k(y, ei_p)


R = 1000


def _dinv_body(dp_ref, dinv_ref):
    deg = jnp.sum(dp_ref[...], axis=0, keepdims=True) + 1.0
    dinv = lax.rsqrt(deg).T
    dinv_ref[...] = jnp.broadcast_to(dinv, dinv_ref.shape)


def _dinv_call(dp):
    return pl.pallas_call(
        _dinv_body,
        out_shape=jax.ShapeDtypeStruct((N_PAD, 8), jnp.float32),
    )(dp)


def _prep_body(dinv8_ref, x0_ref, y1_ref):
    y1_ref[...] = x0_ref[...] * dinv8_ref[:, 0:1]


def _mid_body(p_ref, y1_ref, dinv8_ref, w1_ref, b1_ref, y2_ref):
    dinv = dinv8_ref[:, 0:1]
    ax = (p_ref[0] + p_ref[1] + y1_ref[...]) * dinv
    h = jnp.dot(ax, w1_ref[...], preferred_element_type=jnp.float32)
    x = jnp.maximum(h + b1_ref[...], 0.0)
    y2_ref[...] = x * dinv


def _softmax(v):
    m = jnp.max(v, axis=-1, keepdims=True)
    e = jnp.exp(v - m)
    return e / jnp.sum(e, axis=-1, keepdims=True)


def _head_body(q_ref, y2_ref, dinv8_ref, x0_ref,
               wt_ref, bt_ref, lt_w_ref, lt_b_ref,
               ws_ref, bs_ref, ls_w_ref, ls_b_ref,
               wa_ref, ba_ref, la_w_ref, la_b_ref,
               ltf_w_ref, ltf_b_ref,
               ot_ref, os_ref, otm_ref, oa_ref):
    dinv = dinv8_ref[:, 0:1]
    agg = (q_ref[0] + q_ref[1] + y2_ref[...]) * dinv

    def head(w_ref, b_ref, l_w_ref, l_b_ref):
        c = jnp.dot(agg, w_ref[...], preferred_element_type=jnp.float32)
        c = c + b_ref[...]
        t = jnp.dot(c, l_w_ref[...], preferred_element_type=jnp.float32)
        return _softmax(t + l_b_ref[...])

    ot_ref[...] = head(wt_ref, bt_ref, lt_w_ref, lt_b_ref)
    os_ref[...] = head(ws_ref, bs_ref, ls_w_ref, ls_b_ref)
    oa_ref[...] = head(wa_ref, ba_ref, la_w_ref, la_b_ref)
    tm = jnp.dot(x0_ref[...], ltf_w_ref[...], preferred_element_type=jnp.float32)
    otm_ref[...] = _softmax(tm + ltf_b_ref[...])


def _row_spec(shape):
    nd = len(shape)
    if nd == 3:
        return pl.BlockSpec((shape[0], R, shape[2]), lambda i: (0, i, 0))
    return pl.BlockSpec((R, shape[1]), lambda i: (i, 0))


def _full_spec(shape):
    return pl.BlockSpec(shape, lambda i: tuple(0 for _ in shape))


def _tc_call(body, ins, out_shapes):
    specs = [_row_spec(a.shape) if a.shape[-2] in (N, N_PAD)
             else _full_spec(a.shape) for a in ins]
    out_specs = [_row_spec(s.shape) for s in out_shapes]
    return pl.pallas_call(
        body,
        grid=(N // R,),
        in_specs=specs,
        out_specs=out_specs if len(out_specs) > 1 else out_specs[0],
        out_shape=out_shapes if len(out_shapes) > 1 else out_shapes[0],
    )(*ins)


def kernel(x0, edge_index, W1, b1, Wt, bt, Ws, bs, Wtm, btm, Wa, ba,
           Lt_W, Lt_b, Ls_W, Ls_b, Ltf_W, Ltf_b, La_W, La_b):
    src = edge_index[0].astype(jnp.int32)
    dst = edge_index[1].astype(jnp.int32)
    pad = E_PAD - E_RAW
    pad_i = np.arange(pad, dtype=np.int32)
    src_p = jnp.concatenate([src, jnp.asarray(pad_i % N)]).reshape(-1, C)
    dst_p = jnp.concatenate(
        [dst, jnp.asarray(N + pad_i % (N_PAD - N))]).reshape(-1, C)
    ei_p = jnp.stack([src_p, dst_p], axis=1)

    dp = _deg_kernel(ei_p)
    dinv8 = _dinv_call(dp)
    y1 = _tc_call(_prep_body, [dinv8, x0],
                  [jax.ShapeDtypeStruct((N, D), jnp.float32)])
    p = _agg_kernel(y1, ei_p)
    y2 = _tc_call(_mid_body, [p, y1, dinv8, W1, b1.reshape(1, -1)],
                  [jax.ShapeDtypeStruct((N, D), jnp.float32)])
    q = _agg_kernel(y2, ei_p)
    heads = _tc_call(
        _head_body,
        [q, y2, dinv8, x0,
         Wt, bt.reshape(1, -1), Lt_W, Lt_b.reshape(1, -1),
         Ws, bs.reshape(1, -1), Ls_W, Ls_b.reshape(1, -1),
         Wa, ba.reshape(1, -1), La_W, La_b.reshape(1, -1),
         Ltf_W, Ltf_b.reshape(1, -1)],
        [jax.ShapeDtypeStruct((N, 16), jnp.float32),
         jax.ShapeDtypeStruct((N, 32), jnp.float32),
         jax.ShapeDtypeStruct((N, 8), jnp.float32),
         jax.ShapeDtypeStruct((N, 64), jnp.float32)])
    out_type, out_school, out_time, out_author = heads
    return (out_type, out_school, out_time, out_author)


# speedup vs baseline: 4.0385x; 1.0347x over previous
import dataclasses
import functools

import jax
import jax.numpy as jnp
import numpy as np
from jax import lax
from jax.experimental import pallas as pl
from jax.experimental.pallas import tpu as pltpu
from jax.experimental.pallas import tpu_sc as plsc

N = 10000
D = 128
NC = 2
NS = 16
L = 16
C = 128
E_RAW = 320000
G = 8
NG0 = 10
NG1 = 10
DEPTH = 2
E_PAD = NS * G * C * (NG0 + NG1)
NSTRIPE = -(-N // C)
N_PAD = NSTRIPE * C
DEG_W = D

@functools.lru_cache(maxsize=None)
def _get_mesh():
    return plsc.VectorSubcoreMesh(core_axis_name="c", subcore_axis_name="s",
                                  num_cores=NC, num_subcores=NS)


def _fill(ref, rows, cols, val):
    @pl.loop(0, rows)
    def _(r):
        @pl.loop(0, cols, step=L)
        def _(c0):
            ref[r, pl.ds(c0, L)] = jnp.full((L,), val, jnp.float32)


def _deg_kernel(ei_p):
    @functools.partial(
        pl.kernel,
        out_type=jax.ShapeDtypeStruct((NC * NS, N_PAD), jnp.float32),
        mesh=_get_mesh(),
        scratch_types=[
            pltpu.VMEM((2, G, C), jnp.int32),
            pltpu.VMEM((N_PAD,), jnp.float32),
        ],
        compiler_params=dataclasses.replace(pltpu.CompilerParams(),
                                            needs_layout_passes=False),
    )
    def k(ei_hbm, out_hbm, sdg, acc):
        cid = lax.axis_index("c")
        sid = lax.axis_index("s")
        wid = cid * NS + sid
        ngc = jnp.where(cid == 0, NG0, NG1)
        grp0 = jnp.where(cid == 0, sid * NG0, NS * NG0 + sid * NG1)

        @pl.loop(0, N_PAD, step=L)
        def _(i):
            acc[pl.ds(i, L)] = jnp.zeros((L,), jnp.float32)

        ones = jnp.ones((L,), jnp.float32)

        @pl.loop(0, ngc)
        def _(g):
            crow = (grp0 + g) * G
            pltpu.sync_copy(ei_hbm.at[:, pl.ds(crow, G)], sdg)
            for j in range(G):
                for c0 in range(0, C, L):
                    idx16 = sdg[1, j, pl.ds(c0, L)]
                    plsc.addupdate_scatter(acc, [idx16], ones)

        pltpu.sync_copy(acc, out_hbm.at[wid])

    return k(ei_p)


def _agg_kernel(y, ei_p):
    @functools.partial(
        pl.kernel,
        out_type=jax.ShapeDtypeStruct((NC, N, D), jnp.float32),
        mesh=_get_mesh(),
        scratch_types=[
            pltpu.VMEM((2, G, C), jnp.int32),
        ] + [pltpu.VMEM((C, D), jnp.float32)] * DEPTH + [
            pltpu.VMEM_SHARED((N_PAD, D), jnp.float32),
        ] + [pltpu.SemaphoreType.DMA] * DEPTH,
    )
    def k(y_hbm, ei_hbm, out_hbm, sdg, *rest):
        bufs = rest[:DEPTH]
        zacc = rest[DEPTH]
        sems = rest[DEPTH + 1:]
        cid = lax.axis_index("c")
        sid = lax.axis_index("s")
        ngc = jnp.where(cid == 0, NG0, NG1)
        grp0 = jnp.where(cid == 0, sid * NG0, NS * NG0 + sid * NG1)
        _fill(bufs[0], C, D, 0.0)
        nk = -(-NSTRIPE // NS)
        for kk in range(nk):
            s = jnp.minimum(sid + NS * kk, NSTRIPE - 1)
            pltpu.sync_copy(bufs[0], zacc.at[pl.ds(s * C, C)])
        plsc.subcore_barrier()

        @pl.loop(0, ngc)
        def _(g):
            crow = (grp0 + g) * G
            pltpu.sync_copy(ei_hbm.at[:, pl.ds(crow, G)], sdg)
            hs = [pltpu.async_copy(y_hbm.at[sdg.at[0, j]], bufs[j % DEPTH],
                                   sems[j % DEPTH])
                  for j in range(DEPTH - 1)]
            for j in range(G):
                jn = j + DEPTH - 1
                if jn < G:
                    hs.append(pltpu.async_copy(y_hbm.at[sdg.at[0, jn]],
                                               bufs[jn % DEPTH],
                                               sems[jn % DEPTH]))
                hs[j].wait()
                pltpu.sync_copy(bufs[j % DEPTH], zacc.at[sdg.at[1, j]],
                                add=True)

        plsc.subcore_barrier()
        for kk in range(nk):
            s = jnp.minimum(sid + NS * kk, NSTRIPE - 1)
            start = jnp.minimum(s * C, N - C)
            pltpu.sync_copy(zacc.at[pl.ds(start, C)],
                            out_hbm.at[cid, pl.ds(start, C)])

    return k(y, ei_p)


R = 1000


def _dinv_body(dp_ref, dinv_ref):
    deg = jnp.sum(dp_ref[...], axis=0, keepdims=True) + 1.0
    dinv = lax.rsqrt(deg).T
    dinv_ref[...] = jnp.broadcast_to(dinv, dinv_ref.shape)


def _dinv_call(dp):
    return pl.pallas_call(
        _dinv_body,
        out_shape=jax.ShapeDtypeStruct((N_PAD, 8), jnp.float32),
    )(dp)


def _prep_body(dinv8_ref, x0_ref, y1_ref):
    y1_ref[...] = x0_ref[...] * dinv8_ref[:, 0:1]


def _mid_body(p_ref, y1_ref, dinv8_ref, w1_ref, b1_ref, y2_ref):
    dinv = dinv8_ref[:, 0:1]
    ax = (p_ref[0] + p_ref[1] + y1_ref[...]) * dinv
    h = jnp.dot(ax, w1_ref[...], preferred_element_type=jnp.float32)
    x = jnp.maximum(h + b1_ref[...], 0.0)
    y2_ref[...] = x * dinv


def _softmax(v):
    m = jnp.max(v, axis=-1, keepdims=True)
    e = jnp.exp(v - m)
    return e / jnp.sum(e, axis=-1, keepdims=True)


def _head_body(q_ref, y2_ref, dinv8_ref, x0_ref,
               wt_ref, bt_ref, lt_w_ref, lt_b_ref,
               ws_ref, bs_ref, ls_w_ref, ls_b_ref,
               wa_ref, ba_ref, la_w_ref, la_b_ref,
               ltf_w_ref, ltf_b_ref,
               ot_ref, os_ref, otm_ref, oa_ref):
    dinv = dinv8_ref[:, 0:1]
    agg = (q_ref[0] + q_ref[1] + y2_ref[...]) * dinv

    def head(w_ref, b_ref, l_w_ref, l_b_ref):
        c = jnp.dot(agg, w_ref[...], preferred_element_type=jnp.float32)
        c = c + b_ref[...]
        t = jnp.dot(c, l_w_ref[...], preferred_element_type=jnp.float32)
        return _softmax(t + l_b_ref[...])

    ot_ref[...] = head(wt_ref, bt_ref, lt_w_ref, lt_b_ref)
    os_ref[...] = head(ws_ref, bs_ref, ls_w_ref, ls_b_ref)
    oa_ref[...] = head(wa_ref, ba_ref, la_w_ref, la_b_ref)
    tm = jnp.dot(x0_ref[...], ltf_w_ref[...], preferred_element_type=jnp.float32)
    otm_ref[...] = _softmax(tm + ltf_b_ref[...])


def _row_spec(shape):
    nd = len(shape)
    if nd == 3:
        return pl.BlockSpec((shape[0], R, shape[2]), lambda i: (0, i, 0))
    return pl.BlockSpec((R, shape[1]), lambda i: (i, 0))


def _full_spec(shape):
    return pl.BlockSpec(shape, lambda i: tuple(0 for _ in shape))


def _tc_call(body, ins, out_shapes):
    specs = [_row_spec(a.shape) if a.shape[-2] in (N, N_PAD)
             else _full_spec(a.shape) for a in ins]
    out_specs = [_row_spec(s.shape) for s in out_shapes]
    return pl.pallas_call(
        body,
        grid=(N // R,),
        in_specs=specs,
        out_specs=out_specs if len(out_specs) > 1 else out_specs[0],
        out_shape=out_shapes if len(out_shapes) > 1 else out_shapes[0],
    )(*ins)


def kernel(x0, edge_index, W1, b1, Wt, bt, Ws, bs, Wtm, btm, Wa, ba,
           Lt_W, Lt_b, Ls_W, Ls_b, Ltf_W, Ltf_b, La_W, La_b):
    pad = E_PAD - E_RAW
    pad_i = np.arange(pad, dtype=np.int32)
    pad_c = jnp.asarray(np.stack([pad_i % N, N + pad_i % (N_PAD - N)])
                        .reshape(2, -1, C))
    ei_p = jnp.concatenate(
        [edge_index.astype(jnp.int32).reshape(2, -1, C), pad_c], axis=1)

    dp = _deg_kernel(ei_p)
    dinv8 = _dinv_call(dp)
    y1 = _tc_call(_prep_body, [dinv8, x0],
                  [jax.ShapeDtypeStruct((N, D), jnp.float32)])
    p = _agg_kernel(y1, ei_p)
    y2 = _tc_call(_mid_body, [p, y1, dinv8, W1, b1.reshape(1, -1)],
                  [jax.ShapeDtypeStruct((N, D), jnp.float32)])
    q = _agg_kernel(y2, ei_p)
    heads = _tc_call(
        _head_body,
        [q, y2, dinv8, x0,
         Wt, bt.reshape(1, -1), Lt_W, Lt_b.reshape(1, -1),
         Ws, bs.reshape(1, -1), Ls_W, Ls_b.reshape(1, -1),
         Wa, ba.reshape(1, -1), La_W, La_b.reshape(1, -1),
         Ltf_W, Ltf_b.reshape(1, -1)],
        [jax.ShapeDtypeStruct((N, 16), jnp.float32),
         jax.ShapeDtypeStruct((N, 32), jnp.float32),
         jax.ShapeDtypeStruct((N, 8), jnp.float32),
         jax.ShapeDtypeStruct((N, 64), jnp.float32)])
    out_type, out_school, out_time, out_author = heads
    return (out_type, out_school, out_time, out_author)


# speedup vs baseline: 4.1499x; 1.0276x over previous
import dataclasses
import functools

import jax
import jax.numpy as jnp
import numpy as np
from jax import lax
from jax.experimental import pallas as pl
from jax.experimental.pallas import tpu as pltpu
from jax.experimental.pallas import tpu_sc as plsc

N = 10000
D = 128
NC = 2
NS = 16
L = 16
C = 128
E_RAW = 320000
G = 8
NG0 = 10
NG1 = 10
DEPTH = 2
E_PAD = NS * G * C * (NG0 + NG1)
NSTRIPE = -(-N // C)
N_PAD = NSTRIPE * C
DEG_W = D

@functools.lru_cache(maxsize=None)
def _get_mesh():
    return plsc.VectorSubcoreMesh(core_axis_name="c", subcore_axis_name="s",
                                  num_cores=NC, num_subcores=NS)


def _fill(ref, rows, cols, val):
    @pl.loop(0, rows)
    def _(r):
        @pl.loop(0, cols, step=L)
        def _(c0):
            ref[r, pl.ds(c0, L)] = jnp.full((L,), val, jnp.float32)


def _deg_kernel(ei_p):
    @functools.partial(
        pl.kernel,
        out_type=jax.ShapeDtypeStruct((NC * NS, N_PAD), jnp.float32),
        mesh=_get_mesh(),
        scratch_types=[
            pltpu.VMEM((2, G, C), jnp.int32),
            pltpu.VMEM((N_PAD,), jnp.float32),
        ],
        compiler_params=dataclasses.replace(pltpu.CompilerParams(),
                                            needs_layout_passes=False),
    )
    def k(ei_hbm, out_hbm, sdg, acc):
        cid = lax.axis_index("c")
        sid = lax.axis_index("s")
        wid = cid * NS + sid
        ngc = jnp.where(cid == 0, NG0, NG1)
        grp0 = jnp.where(cid == 0, sid * NG0, NS * NG0 + sid * NG1)

        @pl.loop(0, N_PAD, step=L)
        def _(i):
            acc[pl.ds(i, L)] = jnp.zeros((L,), jnp.float32)

        ones = jnp.ones((L,), jnp.float32)

        @pl.loop(0, ngc)
        def _(g):
            crow = (grp0 + g) * G
            pltpu.sync_copy(ei_hbm.at[:, pl.ds(crow, G)], sdg)
            for j in range(G):
                for c0 in range(0, C, L):
                    idx16 = sdg[1, j, pl.ds(c0, L)]
                    plsc.addupdate_scatter(acc, [idx16], ones)

        pltpu.sync_copy(acc, out_hbm.at[wid])

    return k(ei_p)


def _agg_kernel(y, ei_p):
    @functools.partial(
        pl.kernel,
        out_type=jax.ShapeDtypeStruct((NC, N, D), jnp.float32),
        mesh=_get_mesh(),
        scratch_types=[
            pltpu.VMEM((2, G, C), jnp.int32),
        ] + [pltpu.VMEM((C, D), jnp.float32)] * DEPTH + [
            pltpu.VMEM_SHARED((N_PAD, D), jnp.float32),
        ] + [pltpu.SemaphoreType.DMA] * DEPTH,
    )
    def k(y_hbm, ei_hbm, out_hbm, sdg, *rest):
        bufs = rest[:DEPTH]
        zacc = rest[DEPTH]
        sems = rest[DEPTH + 1:]
        cid = lax.axis_index("c")
        sid = lax.axis_index("s")
        ngc = jnp.where(cid == 0, NG0, NG1)
        grp0 = jnp.where(cid == 0, sid * NG0, NS * NG0 + sid * NG1)
        _fill(bufs[0], C, D, 0.0)
        nk = -(-NSTRIPE // NS)
        for kk in range(nk):
            s = jnp.minimum(sid + NS * kk, NSTRIPE - 1)
            pltpu.sync_copy(bufs[0], zacc.at[pl.ds(s * C, C)])
        plsc.subcore_barrier()

        @pl.loop(0, ngc)
        def _(g):
            crow = (grp0 + g) * G
            pltpu.sync_copy(ei_hbm.at[:, pl.ds(crow, G)], sdg)
            hs = [pltpu.async_copy(y_hbm.at[sdg.at[0, j]], bufs[j % DEPTH],
                                   sems[j % DEPTH])
                  for j in range(DEPTH - 1)]
            for j in range(G):
                jn = j + DEPTH - 1
                if jn < G:
                    hs.append(pltpu.async_copy(y_hbm.at[sdg.at[0, jn]],
                                               bufs[jn % DEPTH],
                                               sems[jn % DEPTH]))
                hs[j].wait()
                pltpu.sync_copy(bufs[j % DEPTH], zacc.at[sdg.at[1, j]],
                                add=True)

        plsc.subcore_barrier()
        for kk in range(nk):
            s = jnp.minimum(sid + NS * kk, NSTRIPE - 1)
            start = jnp.minimum(s * C, N - C)
            pltpu.sync_copy(zacc.at[pl.ds(start, C)],
                            out_hbm.at[cid, pl.ds(start, C)])

    return k(y, ei_p)


R = 2000


def _dinv_body(dp_ref, dinv_ref):
    deg = jnp.sum(dp_ref[...], axis=0, keepdims=True) + 1.0
    dinv = lax.rsqrt(deg).T
    dinv_ref[...] = jnp.broadcast_to(dinv, dinv_ref.shape)


def _dinv_call(dp):
    return pl.pallas_call(
        _dinv_body,
        out_shape=jax.ShapeDtypeStruct((N_PAD, 8), jnp.float32),
    )(dp)


def _prep_body(dinv8_ref, x0_ref, y1_ref):
    y1_ref[...] = x0_ref[...] * dinv8_ref[:, 0:1]


def _mid_body(p_ref, y1_ref, dinv8_ref, w1_ref, b1_ref, y2_ref):
    dinv = dinv8_ref[:, 0:1]
    ax = (p_ref[0] + p_ref[1] + y1_ref[...]) * dinv
    h = jnp.dot(ax, w1_ref[...], preferred_element_type=jnp.float32)
    x = jnp.maximum(h + b1_ref[...], 0.0)
    y2_ref[...] = x * dinv


def _softmax(v):
    m = jnp.max(v, axis=-1, keepdims=True)
    e = jnp.exp(v - m)
    return e / jnp.sum(e, axis=-1, keepdims=True)


def _head_body(q_ref, y2_ref, dinv8_ref, x0_ref,
               wt_ref, bt_ref, lt_w_ref, lt_b_ref,
               ws_ref, bs_ref, ls_w_ref, ls_b_ref,
               wa_ref, ba_ref, la_w_ref, la_b_ref,
               ltf_w_ref, ltf_b_ref,
               ot_ref, os_ref, otm_ref, oa_ref):
    dinv = dinv8_ref[:, 0:1]
    agg = (q_ref[0] + q_ref[1] + y2_ref[...]) * dinv

    def head(w_ref, b_ref, l_w_ref, l_b_ref):
        c = jnp.dot(agg, w_ref[...], preferred_element_type=jnp.float32)
        c = c + b_ref[...]
        t = jnp.dot(c, l_w_ref[...], preferred_element_type=jnp.float32)
        return _softmax(t + l_b_ref[...])

    ot_ref[...] = head(wt_ref, bt_ref, lt_w_ref, lt_b_ref)
    os_ref[...] = head(ws_ref, bs_ref, ls_w_ref, ls_b_ref)
    oa_ref[...] = head(wa_ref, ba_ref, la_w_ref, la_b_ref)
    tm = jnp.dot(x0_ref[...], ltf_w_ref[...], preferred_element_type=jnp.float32)
    otm_ref[...] = _softmax(tm + ltf_b_ref[...])


def _row_spec(shape):
    nd = len(shape)
    if nd == 3:
        return pl.BlockSpec((shape[0], R, shape[2]), lambda i: (0, i, 0))
    return pl.BlockSpec((R, shape[1]), lambda i: (i, 0))


def _full_spec(shape):
    return pl.BlockSpec(shape, lambda i: tuple(0 for _ in shape))


def _tc_call(body, ins, out_shapes):
    specs = [_row_spec(a.shape) if a.shape[-2] in (N, N_PAD)
             else _full_spec(a.shape) for a in ins]
    out_specs = [_row_spec(s.shape) for s in out_shapes]
    return pl.pallas_call(
        body,
        grid=(N // R,),
        in_specs=specs,
        out_specs=out_specs if len(out_specs) > 1 else out_specs[0],
        out_shape=out_shapes if len(out_shapes) > 1 else out_shapes[0],
    )(*ins)


def kernel(x0, edge_index, W1, b1, Wt, bt, Ws, bs, Wtm, btm, Wa, ba,
           Lt_W, Lt_b, Ls_W, Ls_b, Ltf_W, Ltf_b, La_W, La_b):
    pad = E_PAD - E_RAW
    pad_i = np.arange(pad, dtype=np.int32)
    pad_c = jnp.asarray(np.stack([pad_i % N, N + pad_i % (N_PAD - N)])
                        .reshape(2, -1, C))
    ei_p = jnp.concatenate(
        [edge_index.astype(jnp.int32).reshape(2, -1, C), pad_c], axis=1)

    dp = _deg_kernel(ei_p)
    dinv8 = _dinv_call(dp)
    y1 = _tc_call(_prep_body, [dinv8, x0],
                  [jax.ShapeDtypeStruct((N, D), jnp.float32)])
    p = _agg_kernel(y1, ei_p)
    y2 = _tc_call(_mid_body, [p, y1, dinv8, W1, b1.reshape(1, -1)],
                  [jax.ShapeDtypeStruct((N, D), jnp.float32)])
    q = _agg_kernel(y2, ei_p)
    heads = _tc_call(
        _head_body,
        [q, y2, dinv8, x0,
         Wt, bt.reshape(1, -1), Lt_W, Lt_b.reshape(1, -1),
         Ws, bs.reshape(1, -1), Ls_W, Ls_b.reshape(1, -1),
         Wa, ba.reshape(1, -1), La_W, La_b.reshape(1, -1),
         Ltf_W, Ltf_b.reshape(1, -1)],
        [jax.ShapeDtypeStruct((N, 16), jnp.float32),
         jax.ShapeDtypeStruct((N, 32), jnp.float32),
         jax.ShapeDtypeStruct((N, 8), jnp.float32),
         jax.ShapeDtypeStruct((N, 64), jnp.float32)])
    out_type, out_school, out_time, out_author = heads
    return (out_type, out_school, out_time, out_author)
